# baseline (speedup 1.0000x reference)
import functools

import jax
import jax.numpy as jnp
from jax.experimental import pallas as pl
from jax.experimental.pallas import tpu as pltpu


def _fused_kernel(x_ref, w1_ref, b1_ref, w2_ref, b2_ref, e_ref, o_ref, *, hw):
    x = x_ref[...]
    if x.shape[1] != hw:
        lane = jax.lax.broadcasted_iota(jnp.int32, x.shape, 1)
        x_for_max = jnp.where(lane < hw, x, -jnp.inf)
    else:
        x_for_max = x
    s = (jnp.sum(x, axis=1, keepdims=True) * (1.0 / hw)
         + jnp.max(x_for_max, axis=1, keepdims=True))
    h = jnp.dot(w1_ref[...], s, preferred_element_type=jnp.float32) + b1_ref[...]
    h = jnp.maximum(h, 0.0)
    logits = jnp.dot(w2_ref[...], h, preferred_element_type=jnp.float32) + b2_ref[...]
    m = jnp.max(logits, axis=0, keepdims=True)
    p = jnp.exp(logits - m)
    a = p / jnp.sum(p, axis=0, keepdims=True)
    scale = jnp.dot(e_ref[...], a, preferred_element_type=jnp.float32)
    o_ref[...] = (x * scale).astype(o_ref.dtype)


def kernel(x, w1, b1, gamma, beta, run_mean, run_var, w2, b2):
    eps = 1e-5
    B, C, H, W = x.shape
    inter = w1.shape[0]
    groups = w2.shape[0]
    cpg = C // groups
    hw = H * W

    g = gamma / jnp.sqrt(run_var + eps)
    w1e = (w1 * g[:, None]).astype(jnp.float32)
    b1e = (g * (b1 - run_mean) + beta).reshape(inter, 1).astype(jnp.float32)
    w2f = w2.astype(jnp.float32)
    b2c = b2.reshape(groups, 1).astype(jnp.float32)
    E = (jnp.arange(C)[:, None] // cpg == jnp.arange(groups)[None, :]).astype(jnp.float32)

    hwp = (hw + 127) // 128 * 128
    x3 = x.reshape(B, C, hw)
    if hwp != hw:
        x3 = jnp.pad(x3, ((0, 0), (0, 0), (0, hwp - hw)))
    fused = functools.partial(_fused_kernel, hw=hw)
    out = pl.pallas_call(
        fused,
        out_shape=jax.ShapeDtypeStruct((B, C, hwp), x.dtype),
        grid=(B,),
        in_specs=[
            pl.BlockSpec((pl.Squeezed(), C, hwp), lambda b: (b, 0, 0)),
            pl.BlockSpec((inter, C), lambda b: (0, 0)),
            pl.BlockSpec((inter, 1), lambda b: (0, 0)),
            pl.BlockSpec((groups, inter), lambda b: (0, 0)),
            pl.BlockSpec((groups, 1), lambda b: (0, 0)),
            pl.BlockSpec((C, groups), lambda b: (0, 0)),
        ],
        out_specs=pl.BlockSpec((pl.Squeezed(), C, hwp), lambda b: (b, 0, 0)),
        compiler_params=pltpu.CompilerParams(
            dimension_semantics=("parallel",),
            vmem_limit_bytes=48 * 1024 * 1024),
    )(x3, w1e, b1e, w2f, b2c, E)
    if hwp != hw:
        out = out[:, :, :hw]
    return out.reshape(B, C, H, W)


# speedup vs baseline: 1.3195x; 1.3195x over previous
import functools

import jax
import jax.numpy as jnp
from jax.experimental import pallas as pl
from jax.experimental.pallas import tpu as pltpu


def _fused_kernel(x_ref, w1_ref, b1_ref, w2_ref, b2_ref, e_ref, o_ref, *, hw):
    for b in range(x_ref.shape[0]):
        x = x_ref[b]
        s = (jnp.sum(x, axis=1, keepdims=True) * (1.0 / hw)
             + jnp.max(x, axis=1, keepdims=True))
        h = jnp.dot(w1_ref[...], s, preferred_element_type=jnp.float32) + b1_ref[...]
        h = jnp.maximum(h, 0.0)
        logits = jnp.dot(w2_ref[...], h, preferred_element_type=jnp.float32) + b2_ref[...]
        m = jnp.max(logits, axis=0, keepdims=True)
        p = jnp.exp(logits - m)
        a = p / jnp.sum(p, axis=0, keepdims=True)
        scale = jnp.dot(e_ref[...], a, preferred_element_type=jnp.float32)
        o_ref[b] = (x * scale).astype(o_ref.dtype)


def kernel(x, w1, b1, gamma, beta, run_mean, run_var, w2, b2):
    eps = 1e-5
    B, C, H, W = x.shape
    inter = w1.shape[0]
    groups = w2.shape[0]
    cpg = C // groups
    hw = H * W

    g = gamma / jnp.sqrt(run_var + eps)
    w1e = (w1 * g[:, None]).astype(jnp.float32)
    b1e = (g * (b1 - run_mean) + beta).reshape(inter, 1).astype(jnp.float32)
    w2f = w2.astype(jnp.float32)
    b2c = b2.reshape(groups, 1).astype(jnp.float32)
    E = (jnp.arange(C)[:, None] // cpg == jnp.arange(groups)[None, :]).astype(jnp.float32)

    bblk = 4
    x3 = x.reshape(B, C, hw)
    fused = functools.partial(_fused_kernel, hw=hw)
    out = pl.pallas_call(
        fused,
        out_shape=jax.ShapeDtypeStruct((B, C, hw), x.dtype),
        grid=(B // bblk,),
        in_specs=[
            pl.BlockSpec((bblk, C, hw), lambda b: (b, 0, 0)),
            pl.BlockSpec((inter, C), lambda b: (0, 0)),
            pl.BlockSpec((inter, 1), lambda b: (0, 0)),
            pl.BlockSpec((groups, inter), lambda b: (0, 0)),
            pl.BlockSpec((groups, 1), lambda b: (0, 0)),
            pl.BlockSpec((C, groups), lambda b: (0, 0)),
        ],
        out_specs=pl.BlockSpec((bblk, C, hw), lambda b: (b, 0, 0)),
        compiler_params=pltpu.CompilerParams(
            dimension_semantics=("parallel",),
            vmem_limit_bytes=56 * 1024 * 1024),
    )(x3, w1e, b1e, w2f, b2c, E)
    return out.reshape(B, C, H, W)


# speedup vs baseline: 1.3588x; 1.0298x over previous
import functools

import jax
import jax.numpy as jnp
from jax.experimental import pallas as pl
from jax.experimental.pallas import tpu as pltpu


def _fused_kernel(x_ref, w1_ref, b1_ref, w2_ref, b2_ref, e_ref, o_ref, *, hw):
    for b in range(x_ref.shape[0]):
        x = x_ref[b]
        s = (jnp.sum(x, axis=1, keepdims=True) * (1.0 / hw)
             + jnp.max(x, axis=1, keepdims=True))
        h = jnp.dot(w1_ref[...], s, preferred_element_type=jnp.float32) + b1_ref[...]
        h = jnp.maximum(h, 0.0)
        logits = jnp.dot(w2_ref[...], h, preferred_element_type=jnp.float32) + b2_ref[...]
        m = jnp.max(logits, axis=0, keepdims=True)
        p = jnp.exp(logits - m)
        a = p / jnp.sum(p, axis=0, keepdims=True)
        scale = jnp.dot(e_ref[...], a, preferred_element_type=jnp.float32)
        o_ref[b] = (x * scale).astype(o_ref.dtype)


def kernel(x, w1, b1, gamma, beta, run_mean, run_var, w2, b2):
    eps = 1e-5
    B, C, H, W = x.shape
    inter = w1.shape[0]
    groups = w2.shape[0]
    cpg = C // groups
    hw = H * W

    g = gamma / jnp.sqrt(run_var + eps)
    w1e = (w1 * g[:, None]).astype(jnp.float32)
    b1e = (g * (b1 - run_mean) + beta).reshape(inter, 1).astype(jnp.float32)
    w2f = w2.astype(jnp.float32)
    b2c = b2.reshape(groups, 1).astype(jnp.float32)
    E = (jnp.arange(C)[:, None] // cpg == jnp.arange(groups)[None, :]).astype(jnp.float32)

    bblk = 8
    x3 = x.reshape(B, C, hw)
    fused = functools.partial(_fused_kernel, hw=hw)
    out = pl.pallas_call(
        fused,
        out_shape=jax.ShapeDtypeStruct((B, C, hw), x.dtype),
        grid=(B // bblk,),
        in_specs=[
            pl.BlockSpec((bblk, C, hw), lambda b: (b, 0, 0)),
            pl.BlockSpec((inter, C), lambda b: (0, 0)),
            pl.BlockSpec((inter, 1), lambda b: (0, 0)),
            pl.BlockSpec((groups, inter), lambda b: (0, 0)),
            pl.BlockSpec((groups, 1), lambda b: (0, 0)),
            pl.BlockSpec((C, groups), lambda b: (0, 0)),
        ],
        out_specs=pl.BlockSpec((bblk, C, hw), lambda b: (b, 0, 0)),
        compiler_params=pltpu.CompilerParams(
            dimension_semantics=("parallel",),
            vmem_limit_bytes=56 * 1024 * 1024),
    )(x3, w1e, b1e, w2f, b2c, E)
    return out.reshape(B, C, H, W)


# speedup vs baseline: 5.6514x; 4.1591x over previous
import functools

import jax
import jax.numpy as jnp
from jax.experimental import pallas as pl
from jax.experimental.pallas import tpu as pltpu


def _fused_kernel(x_ref, w1_ref, b1_ref, w2_ref, b2_ref, e_ref, o_ref, *, hw):
    x = x_ref[...]
    s = (jnp.sum(x, axis=0) * (1.0 / hw)
         + jnp.max(x, axis=0))
    h = jnp.dot(s, w1_ref[...], preferred_element_type=jnp.float32) + b1_ref[...]
    h = jnp.maximum(h, 0.0)
    logits = jnp.dot(h, w2_ref[...], preferred_element_type=jnp.float32) + b2_ref[...]
    m = jnp.max(logits, axis=1, keepdims=True)
    p = jnp.exp(logits - m)
    a = p / jnp.sum(p, axis=1, keepdims=True)
    scale = jnp.dot(a, e_ref[...], preferred_element_type=jnp.float32)
    o_ref[...] = (x * scale[None, :, :]).astype(o_ref.dtype)


def kernel(x, w1, b1, gamma, beta, run_mean, run_var, w2, b2):
    eps = 1e-5
    B, C, H, W = x.shape
    inter = w1.shape[0]
    groups = w2.shape[0]
    cpg = C // groups
    hw = H * W

    g = gamma / jnp.sqrt(run_var + eps)
    w1t = (w1 * g[:, None]).T.astype(jnp.float32)
    b1r = (g * (b1 - run_mean) + beta).reshape(1, inter).astype(jnp.float32)
    w2t = w2.T.astype(jnp.float32)
    b2r = b2.reshape(1, groups).astype(jnp.float32)
    Et = (jnp.arange(groups)[:, None] == jnp.arange(C)[None, :] // cpg
          ).astype(jnp.float32)

    xt = jnp.transpose(x, (2, 3, 0, 1)).reshape(hw, B, C)
    bblk = next(d for d in (8, 4, 2, 1) if B % d == 0)
    fused = functools.partial(_fused_kernel, hw=hw)
    out = pl.pallas_call(
        fused,
        out_shape=jax.ShapeDtypeStruct((hw, B, C), x.dtype),
        grid=(B // bblk,),
        in_specs=[
            pl.BlockSpec((hw, bblk, C), lambda b: (0, b, 0)),
            pl.BlockSpec((C, inter), lambda b: (0, 0)),
            pl.BlockSpec((1, inter), lambda b: (0, 0)),
            pl.BlockSpec((inter, groups), lambda b: (0, 0)),
            pl.BlockSpec((1, groups), lambda b: (0, 0)),
            pl.BlockSpec((groups, C), lambda b: (0, 0)),
        ],
        out_specs=pl.BlockSpec((hw, bblk, C), lambda b: (0, b, 0)),
        compiler_params=pltpu.CompilerParams(
            dimension_semantics=("parallel",),
            vmem_limit_bytes=60 * 1024 * 1024),
    )(xt, w1t, b1r, w2t, b2r, Et)
    return jnp.transpose(out.reshape(H, W, B, C), (2, 3, 0, 1))


# speedup vs baseline: 5.8839x; 1.0412x over previous
import functools

import jax
import jax.numpy as jnp
from jax.experimental import pallas as pl
from jax.experimental.pallas import tpu as pltpu


def _fused_kernel(x_ref, w1_ref, b1_ref, w2_ref, b2_ref, e_ref, o_ref, *, hw):
    x = x_ref[...]
    s = (jnp.sum(x, axis=0) * (1.0 / hw)
         + jnp.max(x, axis=0))
    h = jnp.dot(s, w1_ref[...], preferred_element_type=jnp.float32) + b1_ref[...]
    h = jnp.maximum(h, 0.0)
    logits = jnp.dot(h, w2_ref[...], preferred_element_type=jnp.float32) + b2_ref[...]
    m = jnp.max(logits, axis=1, keepdims=True)
    p = jnp.exp(logits - m)
    a = p / jnp.sum(p, axis=1, keepdims=True)
    scale = jnp.dot(a, e_ref[...], preferred_element_type=jnp.float32)
    o_ref[...] = (x * scale[None, :, :]).astype(o_ref.dtype)


def kernel(x, w1, b1, gamma, beta, run_mean, run_var, w2, b2):
    eps = 1e-5
    B, C, H, W = x.shape
    inter = w1.shape[0]
    groups = w2.shape[0]
    cpg = C // groups
    hw = H * W

    g = gamma / jnp.sqrt(run_var + eps)
    w1t = (w1 * g[:, None]).T.astype(jnp.float32)
    b1r = (g * (b1 - run_mean) + beta).reshape(1, inter).astype(jnp.float32)
    w2t = w2.T.astype(jnp.float32)
    b2r = b2.reshape(1, groups).astype(jnp.float32)
    Et = (jnp.arange(groups)[:, None] == jnp.arange(C)[None, :] // cpg
          ).astype(jnp.float32)

    xt = jnp.transpose(x, (2, 3, 0, 1)).reshape(hw, B, C)
    bblk = next(d for d in (16, 8, 4, 2, 1) if B % d == 0)
    fused = functools.partial(_fused_kernel, hw=hw)
    out = pl.pallas_call(
        fused,
        out_shape=jax.ShapeDtypeStruct((hw, B, C), x.dtype),
        grid=(B // bblk,),
        in_specs=[
            pl.BlockSpec((hw, bblk, C), lambda b: (0, b, 0)),
            pl.BlockSpec((C, inter), lambda b: (0, 0)),
            pl.BlockSpec((1, inter), lambda b: (0, 0)),
            pl.BlockSpec((inter, groups), lambda b: (0, 0)),
            pl.BlockSpec((1, groups), lambda b: (0, 0)),
            pl.BlockSpec((groups, C), lambda b: (0, 0)),
        ],
        out_specs=pl.BlockSpec((hw, bblk, C), lambda b: (0, b, 0)),
        compiler_params=pltpu.CompilerParams(
            dimension_semantics=("parallel",),
            vmem_limit_bytes=64 * 1024 * 1024),
    )(xt, w1t, b1r, w2t, b2r, Et)
    return jnp.transpose(out.reshape(H, W, B, C), (2, 3, 0, 1))


# speedup vs baseline: 6.2270x; 1.0583x over previous
import functools

import jax
import jax.numpy as jnp
from jax.experimental import pallas as pl
from jax.experimental.pallas import tpu as pltpu


def _contract_last(lhs, rhs):
    return jax.lax.dot_general(lhs, rhs, (((1,), (1,)), ((), ())),
                               preferred_element_type=jnp.float32)


def _fused_kernel(x_ref, w1_ref, g_ref, c_ref, w2_ref, b2_ref, e_ref, o_ref,
                  *, hw):
    x = x_ref[...]
    s = (jnp.sum(x, axis=0) * (1.0 / hw)
         + jnp.max(x, axis=0))
    h = _contract_last(s, w1_ref[...]) * g_ref[...] + c_ref[...]
    h = jnp.maximum(h, 0.0)
    logits = _contract_last(h, w2_ref[...]) + b2_ref[...]
    m = jnp.max(logits, axis=1, keepdims=True)
    p = jnp.exp(logits - m)
    a = p / jnp.sum(p, axis=1, keepdims=True)
    scale = jnp.dot(a, e_ref[...], preferred_element_type=jnp.float32)
    o_ref[...] = (x * scale[None, :, :]).astype(o_ref.dtype)


def kernel(x, w1, b1, gamma, beta, run_mean, run_var, w2, b2):
    eps = 1e-5
    B, C, H, W = x.shape
    inter = w1.shape[0]
    groups = w2.shape[0]
    cpg = C // groups
    hw = H * W

    g = (gamma / jnp.sqrt(run_var + eps)).reshape(1, inter).astype(jnp.float32)
    c = (g * (b1 - run_mean).reshape(1, inter)
         + beta.reshape(1, inter)).astype(jnp.float32)
    b2r = b2.reshape(1, groups).astype(jnp.float32)
    Et = (jnp.arange(groups)[:, None] == jnp.arange(C)[None, :] // cpg
          ).astype(jnp.float32)

    xt = jnp.transpose(x, (2, 3, 0, 1)).reshape(hw, B, C)
    bblk = next(d for d in (16, 8, 4, 2, 1) if B % d == 0)
    fused = functools.partial(_fused_kernel, hw=hw)
    out = pl.pallas_call(
        fused,
        out_shape=jax.ShapeDtypeStruct((hw, B, C), x.dtype),
        grid=(B // bblk,),
        in_specs=[
            pl.BlockSpec((hw, bblk, C), lambda b: (0, b, 0)),
            pl.BlockSpec((inter, C), lambda b: (0, 0)),
            pl.BlockSpec((1, inter), lambda b: (0, 0)),
            pl.BlockSpec((1, inter), lambda b: (0, 0)),
            pl.BlockSpec((groups, inter), lambda b: (0, 0)),
            pl.BlockSpec((1, groups), lambda b: (0, 0)),
            pl.BlockSpec((groups, C), lambda b: (0, 0)),
        ],
        out_specs=pl.BlockSpec((hw, bblk, C), lambda b: (0, b, 0)),
        compiler_params=pltpu.CompilerParams(
            dimension_semantics=("parallel",),
            vmem_limit_bytes=64 * 1024 * 1024),
    )(xt, w1.astype(jnp.float32), g, c, w2.astype(jnp.float32), b2r, Et)
    return jnp.transpose(out.reshape(H, W, B, C), (2, 3, 0, 1))
